# baseline (device time: 44608 ns/iter reference)
import jax
import jax.numpy as jnp
from jax import lax
from jax.experimental import pallas as pl
from jax.experimental.pallas import tpu as pltpu

N_DEV = 4
N_TOK = 2048
D_IN = 512
D_OUT = 1024
N_EXP = 16
EXP_PER_DEV = N_EXP // N_DEV
CAPACITY = 102
CAP_PAD = 104
BLOCK_ROWS = EXP_PER_DEV * CAP_PAD


def kernel(x, router_W, route_idx, expert_W):
    del router_W

    my = lax.axis_index("i")

    e = route_idx[:, 0]
    onehot = (e[:, None] == jnp.arange(N_EXP)[None, :]).astype(jnp.int32)
    ohg = onehot.reshape(N_TOK // 128, 128, N_EXP)
    cg = jnp.cumsum(ohg, axis=1)
    totals = cg[:, -1, :]
    offs = jnp.cumsum(totals, axis=0) - totals
    cum = (cg + offs[:, None, :]).reshape(N_TOK, N_EXP)
    rank = jnp.sum(cum * onehot, axis=1)
    pos = rank - 1
    kept = rank <= CAPACITY

    gslot = jnp.where(
        kept, (e // EXP_PER_DEV) * BLOCK_ROWS + (e % EXP_PER_DEV) * CAP_PAD + pos, -1
    )
    local = gslot - my * BLOCK_ROWS
    slot_local = jnp.where(kept & (local >= 0) & (local < BLOCK_ROWS), local, -1)

    gslot_c = gslot.astype(jnp.int32).reshape(N_TOK, 1)
    slot_row = slot_local.astype(jnp.int32).reshape(1, N_TOK)

    def body(x_ref, w_ref, gslot_ref, slot_row_ref, out_ref,
             own_ref, comm_ref, send_sems, recv_sems):
        my_pos = lax.axis_index("i")
        left = (my_pos - 1) % N_DEV
        right = (my_pos + 1) % N_DEV
        diag = (my_pos + 2) % N_DEV

        barrier_sem = pltpu.get_barrier_semaphore()
        for nbr in [left, right, diag]:
            pl.semaphore_signal(
                barrier_sem, inc=1,
                device_id=(nbr,), device_id_type=pl.DeviceIdType.MESH,
            )
        pl.semaphore_wait(barrier_sem, 3)

        gather_iota = lax.broadcasted_iota(jnp.int32, (BLOCK_ROWS, N_TOK), 0)
        g = (slot_row_ref[...] == gather_iota).astype(jnp.bfloat16)
        xg = x_ref[...].astype(jnp.bfloat16)
        rdmas = {}
        for j in range(EXP_PER_DEV):
            rows = pl.ds(j * CAP_PAD, CAP_PAD)
            xcj = jnp.dot(
                g[j * CAP_PAD : (j + 1) * CAP_PAD], xg,
                preferred_element_type=jnp.float32,
            ).astype(jnp.bfloat16)
            cj = jnp.dot(
                xcj, w_ref[j].astype(jnp.bfloat16),
                preferred_element_type=jnp.float32,
            )
            own_ref[rows, :] = cj.astype(jnp.bfloat16)
            for peer, slot in ((diag, 2), (right, 0), (left, 1)):
                rdma = pltpu.make_async_remote_copy(
                    src_ref=own_ref.at[rows],
                    dst_ref=comm_ref.at[slot, rows],
                    send_sem=send_sems.at[slot * EXP_PER_DEV + j],
                    recv_sem=recv_sems.at[slot * EXP_PER_DEV + j],
                    device_id=(peer,),
                    device_id_type=pl.DeviceIdType.MESH,
                )
                rdma.start()
                rdmas[(slot, j)] = rdma

        scat_iota = lax.broadcasted_iota(jnp.int32, (N_TOK, BLOCK_ROWS), 1)
        gslot_v = gslot_ref[...]

        def scatter_block(origin, block_ref):
            p = (gslot_v == scat_iota + origin * BLOCK_ROWS).astype(jnp.bfloat16)
            contrib = jnp.dot(p, block_ref[...], preferred_element_type=jnp.float32)
            out_ref[...] += contrib.astype(jnp.bfloat16)

        own_contrib = lax.dot_general(
            g, own_ref[...], (((0,), (0,)), ((), ())),
            preferred_element_type=jnp.float32,
        )
        out_ref[...] = own_contrib.astype(jnp.bfloat16)
        for origin, slot in ((left, 0), (right, 1), (diag, 2)):
            for j in range(EXP_PER_DEV):
                rdmas[(slot, j)].wait_recv()
            scatter_block(origin, comm_ref.at[slot])

        for rdma in rdmas.values():
            rdma.wait_send()

    return pl.pallas_call(
        body,
        out_shape=jax.ShapeDtypeStruct((N_TOK, D_OUT), jnp.bfloat16),
        in_specs=[pl.BlockSpec(memory_space=pltpu.VMEM)] * 4,
        out_specs=pl.BlockSpec(memory_space=pltpu.VMEM),
        scratch_shapes=[
            pltpu.VMEM((BLOCK_ROWS, D_OUT), jnp.bfloat16),
            pltpu.VMEM((N_DEV - 1, BLOCK_ROWS, D_OUT), jnp.bfloat16),
            pltpu.SemaphoreType.DMA(((N_DEV - 1) * EXP_PER_DEV,)),
            pltpu.SemaphoreType.DMA(((N_DEV - 1) * EXP_PER_DEV,)),
        ],
        compiler_params=pltpu.CompilerParams(collective_id=0),
    )(x, expert_W, gslot_c, slot_row)


# device time: 44191 ns/iter; 1.0094x vs baseline; 1.0094x over previous
import jax
import jax.numpy as jnp
from jax import lax
from jax.experimental import pallas as pl
from jax.experimental.pallas import tpu as pltpu

N_DEV = 4
N_TOK = 2048
D_IN = 512
D_OUT = 1024
N_EXP = 16
EXP_PER_DEV = N_EXP // N_DEV
CAPACITY = 102
CAP_PAD = 104
BLOCK_ROWS = EXP_PER_DEV * CAP_PAD


def kernel(x, router_W, route_idx, expert_W):
    del router_W

    my = lax.axis_index("i")

    e = route_idx[:, 0]
    onehot = (e[:, None] == jnp.arange(N_EXP)[None, :]).astype(jnp.int32)
    ohg = onehot.reshape(N_TOK // 128, 128, N_EXP)
    cg = jnp.cumsum(ohg, axis=1)
    totals = cg[:, -1, :]
    offs = jnp.cumsum(totals, axis=0) - totals
    cum = (cg + offs[:, None, :]).reshape(N_TOK, N_EXP)
    rank = jnp.sum(cum * onehot, axis=1)
    pos = rank - 1
    kept = rank <= CAPACITY

    gslot = jnp.where(
        kept, (e // EXP_PER_DEV) * BLOCK_ROWS + (e % EXP_PER_DEV) * CAP_PAD + pos, -1
    )
    local = gslot - my * BLOCK_ROWS
    slot_local = jnp.where(kept & (local >= 0) & (local < BLOCK_ROWS), local, -1)

    gslot_c = gslot.astype(jnp.int32).reshape(N_TOK, 1)
    slot_c = slot_local.astype(jnp.int32).reshape(N_TOK, 1)

    def body(x_ref, w_ref, gslot_ref, slot_c_ref, out_ref,
             own_ref, comm_ref, send_sems, recv_sems):
        my_pos = lax.axis_index("i")
        left = (my_pos - 1) % N_DEV
        right = (my_pos + 1) % N_DEV
        diag = (my_pos + 2) % N_DEV

        barrier_sem = pltpu.get_barrier_semaphore()
        for nbr in [left, right, diag]:
            pl.semaphore_signal(
                barrier_sem, inc=1,
                device_id=(nbr,), device_id_type=pl.DeviceIdType.MESH,
            )
        pl.semaphore_wait(barrier_sem, 3)

        slot_iota = lax.broadcasted_iota(jnp.int32, (N_TOK, BLOCK_ROWS), 1)
        gt = (slot_c_ref[...] == slot_iota).astype(jnp.bfloat16)
        xg = x_ref[...].astype(jnp.bfloat16)
        rdmas = {}
        for j in range(EXP_PER_DEV):
            rows = pl.ds(j * CAP_PAD, CAP_PAD)
            xcj = lax.dot_general(
                gt[:, j * CAP_PAD : (j + 1) * CAP_PAD], xg,
                (((0,), (0,)), ((), ())),
                preferred_element_type=jnp.float32,
            ).astype(jnp.bfloat16)
            cj = jnp.dot(
                xcj, w_ref[j].astype(jnp.bfloat16),
                preferred_element_type=jnp.float32,
            )
            own_ref[rows, :] = cj.astype(jnp.bfloat16)
            for peer, slot in ((diag, 2), (right, 0), (left, 1)):
                rdma = pltpu.make_async_remote_copy(
                    src_ref=own_ref.at[rows],
                    dst_ref=comm_ref.at[slot, rows],
                    send_sem=send_sems.at[slot * EXP_PER_DEV + j],
                    recv_sem=recv_sems.at[slot * EXP_PER_DEV + j],
                    device_id=(peer,),
                    device_id_type=pl.DeviceIdType.MESH,
                )
                rdma.start()
                rdmas[(slot, j)] = rdma

        gslot_v = gslot_ref[...]

        def scatter_block(origin, block_ref):
            p = (gslot_v == slot_iota + origin * BLOCK_ROWS).astype(jnp.bfloat16)
            contrib = jnp.dot(p, block_ref[...], preferred_element_type=jnp.float32)
            out_ref[...] += contrib.astype(jnp.bfloat16)

        own_contrib = jnp.dot(
            gt, own_ref[...], preferred_element_type=jnp.float32
        )
        out_ref[...] = own_contrib.astype(jnp.bfloat16)
        for origin, slot in ((left, 0), (right, 1), (diag, 2)):
            for j in range(EXP_PER_DEV):
                rdmas[(slot, j)].wait_recv()
            scatter_block(origin, comm_ref.at[slot])

        for rdma in rdmas.values():
            rdma.wait_send()

    return pl.pallas_call(
        body,
        out_shape=jax.ShapeDtypeStruct((N_TOK, D_OUT), jnp.bfloat16),
        in_specs=[pl.BlockSpec(memory_space=pltpu.VMEM)] * 4,
        out_specs=pl.BlockSpec(memory_space=pltpu.VMEM),
        scratch_shapes=[
            pltpu.VMEM((BLOCK_ROWS, D_OUT), jnp.bfloat16),
            pltpu.VMEM((N_DEV - 1, BLOCK_ROWS, D_OUT), jnp.bfloat16),
            pltpu.SemaphoreType.DMA(((N_DEV - 1) * EXP_PER_DEV,)),
            pltpu.SemaphoreType.DMA(((N_DEV - 1) * EXP_PER_DEV,)),
        ],
        compiler_params=pltpu.CompilerParams(collective_id=0),
    )(x, expert_W, gslot_c, slot_c)
